# baseline (speedup 1.0000x reference)
import functools

import jax
import jax.numpy as jnp
from jax import lax
from jax.experimental import pallas as pl
from jax.experimental.pallas import tpu as pltpu
from jax.experimental.pallas import tpu_sc as plsc

N = 10000
F = 128
H = 8
HID = 16
NEG = 0.2
NP = 10112
E = 320000
EP = E + N
K = 96
NW = 32
BLOCKS = 4 * (-(-EP // (4 * NW * K)))
NB4 = BLOCKS // 4
EPAD = BLOCKS * NW * K
PER_W = BLOCKS * K
RPT = NP // 16
ZR = 32

_f32 = jnp.float32


def _front_body(x_ref, w_ref, asm_ref, adm_ref, h_ref, as_ref, ad_ref):
    h = jnp.dot(x_ref[...], w_ref[...], preferred_element_type=_f32)
    h_ref[pl.ds(0, N), :] = h
    h_ref[pl.ds(N, NP - N), :] = jnp.zeros((NP - N, F), _f32)
    as_ref[pl.ds(0, N), :] = jnp.dot(h, asm_ref[...], preferred_element_type=_f32)
    as_ref[pl.ds(N, NP - N), :] = jnp.zeros((NP - N, 16), _f32)
    ad_ref[pl.ds(0, N), :] = jnp.dot(h, adm_ref[...], preferred_element_type=_f32)
    ad_ref[pl.ds(N, NP - N), :] = jnp.zeros((NP - N, 16), _f32)


_tc_front = pl.pallas_call(
    _front_body,
    out_shape=(
        jax.ShapeDtypeStruct((NP, F), _f32),
        jax.ShapeDtypeStruct((NP, 16), _f32),
        jax.ShapeDtypeStruct((NP, 16), _f32),
    ),
)


def _mid_body(np_ref, dp_ref, b_ref, g_ref, be_ref, w_ref, asm_ref, adm_ref,
              p_ref, h_ref, as_ref, ad_ref):
    num = np_ref[0] + np_ref[1]
    den = dp_ref[0] + dp_ref[1]
    den_e = jnp.dot(den[:N], p_ref[...], preferred_element_type=_f32) + 1e-16
    out0 = num[:N] / den_e + b_ref[...]
    m = jnp.mean(out0, axis=0)
    v = jnp.mean((out0 - m) ** 2, axis=0)
    act = g_ref[...] * (out0 - m) / jnp.sqrt(v + 1e-5) + be_ref[...]
    act = jnp.where(act > 0, act, jnp.exp(act) - 1.0)
    h1 = jnp.dot(act, w_ref[...], preferred_element_type=_f32)
    h_ref[pl.ds(0, N), :] = h1
    h_ref[pl.ds(N, NP - N), :] = jnp.zeros((NP - N, F), _f32)
    as_ref[pl.ds(0, N), :] = jnp.dot(h1, asm_ref[...], preferred_element_type=_f32)
    as_ref[pl.ds(N, NP - N), :] = jnp.zeros((NP - N, 16), _f32)
    ad_ref[pl.ds(0, N), :] = jnp.dot(h1, adm_ref[...], preferred_element_type=_f32)
    ad_ref[pl.ds(N, NP - N), :] = jnp.zeros((NP - N, 16), _f32)


_tc_mid = pl.pallas_call(
    _mid_body,
    out_shape=(
        jax.ShapeDtypeStruct((NP, F), _f32),
        jax.ShapeDtypeStruct((NP, 16), _f32),
        jax.ShapeDtypeStruct((NP, 16), _f32),
    ),
)


def _final_body(np_ref, dp_ref, b_ref, g_ref, be_ref, p_ref, q_ref, wc_ref,
                bc_ref, out_ref):
    num = np_ref[0] + np_ref[1]
    den = dp_ref[0] + dp_ref[1]
    den_e = jnp.dot(den[:N], p_ref[...], preferred_element_type=_f32) + 1e-16
    full = num[:N] / den_e
    mean16 = jnp.dot(full, q_ref[...], preferred_element_type=_f32) + b_ref[...]
    m = jnp.mean(mean16, axis=0)
    v = jnp.mean((mean16 - m) ** 2, axis=0)
    act = g_ref[...] * (mean16 - m) / jnp.sqrt(v + 1e-5) + be_ref[...]
    out_ref[...] = jnp.dot(act, wc_ref[...], preferred_element_type=_f32) + bc_ref[...]


_tc_final = pl.pallas_call(
    _final_body,
    out_shape=jax.ShapeDtypeStruct((N, F), _f32),
)


def _lane_bcast(v, h):
    idx = jnp.full((16,), h, jnp.int32)
    dn = lax.GatherDimensionNumbers(
        offset_dims=(), collapsed_slice_dims=(0,), start_index_map=(0,))
    return lax.gather(v, idx[:, None], dn, (1,),
                      mode=lax.GatherScatterMode.PROMISE_IN_BOUNDS)


def _sc_agg_body(h_hbm, as_hbm, ad_hbm, src_hbm, dst_hbm,
                 num_out, den_out,
                 num_sh, den_sh, srcq, dstq, rows, aSv, aDv, wv, zb, zbd,
                 semH0, semH1, semA0, semA1, semD0, semD1,
                 semI0, semI1, semI2, semI3):
    semH = (semH0, semH1)
    semA = (semA0, semA1)
    semD = (semD0, semD1)
    semI = (semI0, semI1, semI2, semI3)
    cid = lax.axis_index("c")
    sid = lax.axis_index("s")
    wid = sid * 2 + cid

    zero16 = jnp.zeros((16,), _f32)

    def zrow(i, _):
        for j in range(F // 16):
            zb[i, pl.ds(j * 16, 16)] = zero16
        zbd[i, :] = zero16
        return 0

    lax.fori_loop(0, ZR, zrow, 0)
    rstart = sid * RPT
    nfull, rem = divmod(RPT, ZR)
    for rep in range(nfull):
        pltpu.async_copy(zb, num_sh.at[pl.ds(rstart + rep * ZR, ZR)], semI0)
        pltpu.async_copy(zbd, den_sh.at[pl.ds(rstart + rep * ZR, ZR)], semI0)
    if rem:
        pltpu.async_copy(zb.at[pl.ds(0, rem)],
                         num_sh.at[pl.ds(rstart + nfull * ZR, rem)], semI0)
        pltpu.async_copy(zbd.at[pl.ds(0, rem)],
                         den_sh.at[pl.ds(rstart + nfull * ZR, rem)], semI0)
    for rep in range(nfull):
        pltpu.make_async_copy(zb, num_sh.at[pl.ds(rstart, ZR)], semI0).wait()
        pltpu.make_async_copy(zbd, den_sh.at[pl.ds(rstart, ZR)], semI0).wait()
    if rem:
        pltpu.make_async_copy(zb.at[pl.ds(0, rem)],
                              num_sh.at[pl.ds(rstart, rem)], semI0).wait()
        pltpu.make_async_copy(zbd.at[pl.ds(0, rem)],
                              den_sh.at[pl.ds(rstart, rem)], semI0).wait()
    plsc.subcore_barrier()

    def issue_idx(s, g):
        base = wid * PER_W + g * K
        pltpu.async_copy(src_hbm.at[pl.ds(base, K)], srcq.at[s], semI[s])
        pltpu.async_copy(dst_hbm.at[pl.ds(base, K)], dstq.at[s], semI[s])

    def wait_idx(s):
        pltpu.make_async_copy(src_hbm.at[pl.ds(0, K)], srcq.at[s], semI[s]).wait()
        pltpu.make_async_copy(dst_hbm.at[pl.ds(0, K)], dstq.at[s], semI[s]).wait()

    def issue_gathers(p, s):
        pltpu.async_copy(h_hbm.at[srcq.at[s]], rows.at[p], semH[p])
        pltpu.async_copy(as_hbm.at[srcq.at[s]], aSv.at[p], semA[p])
        pltpu.async_copy(ad_hbm.at[dstq.at[s]], aDv.at[p], semD[p])

    def wait_gathers(p, s):
        pltpu.make_async_copy(h_hbm.at[srcq.at[s]], rows.at[p], semH[p]).wait()
        pltpu.make_async_copy(as_hbm.at[srcq.at[s]], aSv.at[p], semA[p]).wait()
        pltpu.make_async_copy(ad_hbm.at[dstq.at[s]], aDv.at[p], semD[p]).wait()

    def compute(p, s):
        rp, ap, dp, wp = rows.at[p], aSv.at[p], aDv.at[p], wv.at[p]

        @plsc.parallel_loop(0, K, 1, unroll=4)
        def edge(e):
            ev = ap[e, :] + dp[e, :]
            ev = jnp.where(ev >= 0, ev, NEG * ev)
            w = jnp.exp(ev)
            wp[e, :] = w
            for h in range(H):
                wb = _lane_bcast(w, h)
                rp[e, pl.ds(h * HID, HID)] = rp[e, pl.ds(h * HID, HID)] * wb

        pltpu.sync_copy(rp, num_sh.at[dstq.at[s]], add=True)
        pltpu.sync_copy(wp, den_sh.at[dstq.at[s]], add=True)

    issue_idx(0, 0)
    issue_idx(1, 1)
    issue_idx(2, 2)
    wait_idx(0)
    issue_gathers(0, 0)

    def body(b4, _):
        g0 = b4 * 4
        for j in range(4):
            wait_gathers(j % 2, j)
            if j == 3:
                @pl.when(b4 < NB4 - 1)
                def _():
                    wait_idx(0)
                    issue_gathers(0, 0)
            else:
                wait_idx(j + 1)
                issue_gathers((j + 1) % 2, j + 1)
            if j == 0:
                issue_idx(3, g0 + 3)
            else:
                @pl.when(b4 < NB4 - 1)
                def _():
                    issue_idx((j + 3) % 4, g0 + j + 3)
            compute(j % 2, j)
        return 0

    lax.fori_loop(0, NB4, body, 0)
    plsc.subcore_barrier()

    pltpu.sync_copy(num_sh.at[pl.ds(rstart, RPT)],
                    num_out.at[cid, pl.ds(rstart, RPT)])
    pltpu.sync_copy(den_sh.at[pl.ds(rstart, RPT)],
                    den_out.at[cid, pl.ds(rstart, RPT)])


@functools.cache
def _make_sc_agg():
    return pl.kernel(
        _sc_agg_body,
        out_type=(
            jax.ShapeDtypeStruct((2, NP, F), _f32),
            jax.ShapeDtypeStruct((2, NP, 16), _f32),
        ),
        mesh=plsc.VectorSubcoreMesh(core_axis_name="c", subcore_axis_name="s"),
        compiler_params=pltpu.CompilerParams(use_tc_tiling_on_sc=False),
        scratch_types=[
            pltpu.VMEM_SHARED((NP, F), _f32),
            pltpu.VMEM_SHARED((NP, 16), _f32),
            pltpu.VMEM((4, K), jnp.int32),
            pltpu.VMEM((4, K), jnp.int32),
            pltpu.VMEM((2, K, F), _f32),
            pltpu.VMEM((2, K, 16), _f32),
            pltpu.VMEM((2, K, 16), _f32),
            pltpu.VMEM((2, K, 16), _f32),
            pltpu.VMEM((ZR, F), _f32),
            pltpu.VMEM((ZR, 16), _f32),
        ] + [pltpu.SemaphoreType.DMA] * 10,
    )


def _att_mats(a_src, a_dst):
    eye = jnp.eye(H, dtype=_f32)
    asm = (a_src[:, :, None] * eye[:, None, :]).reshape(F, H)
    adm = (a_dst[:, :, None] * eye[:, None, :]).reshape(F, H)
    pad = jnp.zeros((F, 16 - H), _f32)
    return jnp.concatenate([asm, pad], axis=1), jnp.concatenate([adm, pad], axis=1)


def kernel(x, edge_index, W0, as0, ad0, b0, g0, be0, W1, as1, ad1, b1, g1, be1, Wc, bc):
    asm0, adm0 = _att_mats(as0, ad0)
    asm1, adm1 = _att_mats(as1, ad1)
    P = jnp.concatenate(
        [jnp.repeat(jnp.eye(H, dtype=_f32), HID, axis=1),
         jnp.zeros((16 - H, F), _f32)], axis=0)
    Q = jnp.tile(jnp.eye(HID, dtype=_f32) / H, (H, 1))
    Wcp = jnp.concatenate([Wc, jnp.zeros((HID, F - Wc.shape[1]), _f32)], axis=1)
    bcp = jnp.concatenate([bc, jnp.zeros((F - bc.shape[0],), _f32)])

    loops = jnp.arange(N, dtype=jnp.int32)
    padi = jnp.full((EPAD - EP,), N, jnp.int32)
    src2 = jnp.concatenate([edge_index[0].astype(jnp.int32), loops, padi])
    dst2 = jnp.concatenate([edge_index[1].astype(jnp.int32), loops, padi])

    sc_agg = _make_sc_agg()
    h0, aS0, aD0 = _tc_front(x, W0, asm0, adm0)
    n0, d0 = sc_agg(h0, aS0, aD0, src2, dst2)
    h1, aS1, aD1 = _tc_mid(n0, d0, b0, g0, be0, W1, asm1, adm1, P)
    n1, d1 = sc_agg(h1, aS1, aD1, src2, dst2)
    out = _tc_final(n1, d1, b1, g1, be1, P, Q, Wcp, bcp)
    return out[:, :2]


# speedup vs baseline: 1.0093x; 1.0093x over previous
import functools

import jax
import jax.numpy as jnp
from jax import lax
from jax.experimental import pallas as pl
from jax.experimental.pallas import tpu as pltpu
from jax.experimental.pallas import tpu_sc as plsc

N = 10000
F = 128
H = 8
HID = 16
NEG = 0.2
NP = 10112
E = 320000
EP = E + N
K = 96
NW = 32
BLOCKS = 4 * (-(-EP // (4 * NW * K)))
NB4 = BLOCKS // 4
EPAD = BLOCKS * NW * K
PER_W = BLOCKS * K
RPT = NP // 16
ZR = 32

_f32 = jnp.float32


def _front_body(x_ref, w_ref, asm_ref, adm_ref, h_ref, as_ref, ad_ref):
    h = jnp.dot(x_ref[...], w_ref[...], preferred_element_type=_f32)
    h_ref[pl.ds(0, N), :] = h
    h_ref[pl.ds(N, NP - N), :] = jnp.zeros((NP - N, F), _f32)
    as_ref[pl.ds(0, N), :] = jnp.dot(h, asm_ref[...], preferred_element_type=_f32)
    as_ref[pl.ds(N, NP - N), :] = jnp.zeros((NP - N, 16), _f32)
    ad_ref[pl.ds(0, N), :] = jnp.dot(h, adm_ref[...], preferred_element_type=_f32)
    ad_ref[pl.ds(N, NP - N), :] = jnp.zeros((NP - N, 16), _f32)


_tc_front = pl.pallas_call(
    _front_body,
    out_shape=(
        jax.ShapeDtypeStruct((NP, F), _f32),
        jax.ShapeDtypeStruct((NP, 16), _f32),
        jax.ShapeDtypeStruct((NP, 16), _f32),
    ),
)


def _mid_body(np_ref, dp_ref, b_ref, g_ref, be_ref, w_ref, asm_ref, adm_ref,
              p_ref, h_ref, as_ref, ad_ref):
    num = np_ref[0] + np_ref[1]
    den = dp_ref[0] + dp_ref[1]
    den_e = jnp.dot(den[:N], p_ref[...], preferred_element_type=_f32) + 1e-16
    out0 = num[:N] / den_e + b_ref[...]
    m = jnp.mean(out0, axis=0)
    v = jnp.mean((out0 - m) ** 2, axis=0)
    act = g_ref[...] * (out0 - m) / jnp.sqrt(v + 1e-5) + be_ref[...]
    act = jnp.where(act > 0, act, jnp.exp(act) - 1.0)
    h1 = jnp.dot(act, w_ref[...], preferred_element_type=_f32)
    h_ref[pl.ds(0, N), :] = h1
    h_ref[pl.ds(N, NP - N), :] = jnp.zeros((NP - N, F), _f32)
    as_ref[pl.ds(0, N), :] = jnp.dot(h1, asm_ref[...], preferred_element_type=_f32)
    as_ref[pl.ds(N, NP - N), :] = jnp.zeros((NP - N, 16), _f32)
    ad_ref[pl.ds(0, N), :] = jnp.dot(h1, adm_ref[...], preferred_element_type=_f32)
    ad_ref[pl.ds(N, NP - N), :] = jnp.zeros((NP - N, 16), _f32)


_tc_mid = pl.pallas_call(
    _mid_body,
    out_shape=(
        jax.ShapeDtypeStruct((NP, F), _f32),
        jax.ShapeDtypeStruct((NP, 16), _f32),
        jax.ShapeDtypeStruct((NP, 16), _f32),
    ),
)


def _final_body(np_ref, dp_ref, b_ref, g_ref, be_ref, p_ref, q_ref, wc_ref,
                bc_ref, out_ref):
    num = np_ref[0] + np_ref[1]
    den = dp_ref[0] + dp_ref[1]
    den_e = jnp.dot(den[:N], p_ref[...], preferred_element_type=_f32) + 1e-16
    full = num[:N] / den_e
    mean16 = jnp.dot(full, q_ref[...], preferred_element_type=_f32) + b_ref[...]
    m = jnp.mean(mean16, axis=0)
    v = jnp.mean((mean16 - m) ** 2, axis=0)
    act = g_ref[...] * (mean16 - m) / jnp.sqrt(v + 1e-5) + be_ref[...]
    out_ref[...] = jnp.dot(act, wc_ref[...], preferred_element_type=_f32) + bc_ref[...]


_tc_final = pl.pallas_call(
    _final_body,
    out_shape=jax.ShapeDtypeStruct((N, F), _f32),
)


def _lane_bcast(v, h):
    idx = jnp.full((16,), h, jnp.int32)
    dn = lax.GatherDimensionNumbers(
        offset_dims=(), collapsed_slice_dims=(0,), start_index_map=(0,))
    return lax.gather(v, idx[:, None], dn, (1,),
                      mode=lax.GatherScatterMode.PROMISE_IN_BOUNDS)


def _sc_agg_body(h_hbm, as_hbm, ad_hbm, src_hbm, dst_hbm,
                 num_out, den_out,
                 num_sh, den_sh, srcq, dstq, rows, aSv, aDv, wv, zb, zbd,
                 semH0, semH1, semA0, semA1, semD0, semD1,
                 semI0, semI1, semI2, semI3, semZ):
    semH = (semH0, semH1)
    semA = (semA0, semA1)
    semD = (semD0, semD1)
    semI = (semI0, semI1, semI2, semI3)
    cid = lax.axis_index("c")
    sid = lax.axis_index("s")
    wid = sid * 2 + cid

    zero16 = jnp.zeros((16,), _f32)

    def zrow(i, _):
        for j in range(F // 16):
            zb[i, pl.ds(j * 16, 16)] = zero16
        zbd[i, :] = zero16
        return 0

    lax.fori_loop(0, ZR, zrow, 0)
    rstart = sid * RPT
    nfull, rem = divmod(RPT, ZR)
    for rep in range(nfull):
        pltpu.async_copy(zb, num_sh.at[pl.ds(rstart + rep * ZR, ZR)], semZ)
        pltpu.async_copy(zbd, den_sh.at[pl.ds(rstart + rep * ZR, ZR)], semZ)
    if rem:
        pltpu.async_copy(zb.at[pl.ds(0, rem)],
                         num_sh.at[pl.ds(rstart + nfull * ZR, rem)], semZ)
        pltpu.async_copy(zbd.at[pl.ds(0, rem)],
                         den_sh.at[pl.ds(rstart + nfull * ZR, rem)], semZ)

    def issue_idx(s, g):
        base = wid * PER_W + g * K
        pltpu.async_copy(src_hbm.at[pl.ds(base, K)], srcq.at[s], semI[s])
        pltpu.async_copy(dst_hbm.at[pl.ds(base, K)], dstq.at[s], semI[s])

    def wait_idx(s):
        pltpu.make_async_copy(src_hbm.at[pl.ds(0, K)], srcq.at[s], semI[s]).wait()
        pltpu.make_async_copy(dst_hbm.at[pl.ds(0, K)], dstq.at[s], semI[s]).wait()

    def issue_gathers(p, s):
        pltpu.async_copy(h_hbm.at[srcq.at[s]], rows.at[p], semH[p])
        pltpu.async_copy(as_hbm.at[srcq.at[s]], aSv.at[p], semA[p])
        pltpu.async_copy(ad_hbm.at[dstq.at[s]], aDv.at[p], semD[p])

    def wait_gathers(p, s):
        pltpu.make_async_copy(h_hbm.at[srcq.at[s]], rows.at[p], semH[p]).wait()
        pltpu.make_async_copy(as_hbm.at[srcq.at[s]], aSv.at[p], semA[p]).wait()
        pltpu.make_async_copy(ad_hbm.at[dstq.at[s]], aDv.at[p], semD[p]).wait()

    def compute(p, s):
        rp, ap, dp, wp = rows.at[p], aSv.at[p], aDv.at[p], wv.at[p]

        @plsc.parallel_loop(0, K, 1, unroll=4)
        def edge(e):
            ev = ap[e, :] + dp[e, :]
            ev = jnp.where(ev >= 0, ev, NEG * ev)
            w = jnp.exp(ev)
            wp[e, :] = w
            for h in range(H):
                wb = _lane_bcast(w, h)
                rp[e, pl.ds(h * HID, HID)] = rp[e, pl.ds(h * HID, HID)] * wb

        pltpu.sync_copy(rp, num_sh.at[dstq.at[s]], add=True)
        pltpu.sync_copy(wp, den_sh.at[dstq.at[s]], add=True)

    issue_idx(0, 0)
    issue_idx(1, 1)
    issue_idx(2, 2)
    wait_idx(0)
    issue_gathers(0, 0)
    for rep in range(nfull):
        pltpu.make_async_copy(zb, num_sh.at[pl.ds(rstart, ZR)], semZ).wait()
        pltpu.make_async_copy(zbd, den_sh.at[pl.ds(rstart, ZR)], semZ).wait()
    if rem:
        pltpu.make_async_copy(zb.at[pl.ds(0, rem)],
                              num_sh.at[pl.ds(rstart, rem)], semZ).wait()
        pltpu.make_async_copy(zbd.at[pl.ds(0, rem)],
                              den_sh.at[pl.ds(rstart, rem)], semZ).wait()
    plsc.subcore_barrier()

    def body(b4, _):
        g0 = b4 * 4
        for j in range(4):
            wait_gathers(j % 2, j)
            if j == 3:
                @pl.when(b4 < NB4 - 1)
                def _():
                    wait_idx(0)
                    issue_gathers(0, 0)
            else:
                wait_idx(j + 1)
                issue_gathers((j + 1) % 2, j + 1)
            if j == 0:
                issue_idx(3, g0 + 3)
            else:
                @pl.when(b4 < NB4 - 1)
                def _():
                    issue_idx((j + 3) % 4, g0 + j + 3)
            compute(j % 2, j)
        return 0

    lax.fori_loop(0, NB4, body, 0)
    plsc.subcore_barrier()

    pltpu.sync_copy(num_sh.at[pl.ds(rstart, RPT)],
                    num_out.at[cid, pl.ds(rstart, RPT)])
    pltpu.sync_copy(den_sh.at[pl.ds(rstart, RPT)],
                    den_out.at[cid, pl.ds(rstart, RPT)])


@functools.cache
def _make_sc_agg():
    return pl.kernel(
        _sc_agg_body,
        out_type=(
            jax.ShapeDtypeStruct((2, NP, F), _f32),
            jax.ShapeDtypeStruct((2, NP, 16), _f32),
        ),
        mesh=plsc.VectorSubcoreMesh(core_axis_name="c", subcore_axis_name="s"),
        compiler_params=pltpu.CompilerParams(use_tc_tiling_on_sc=False),
        scratch_types=[
            pltpu.VMEM_SHARED((NP, F), _f32),
            pltpu.VMEM_SHARED((NP, 16), _f32),
            pltpu.VMEM((4, K), jnp.int32),
            pltpu.VMEM((4, K), jnp.int32),
            pltpu.VMEM((2, K, F), _f32),
            pltpu.VMEM((2, K, 16), _f32),
            pltpu.VMEM((2, K, 16), _f32),
            pltpu.VMEM((2, K, 16), _f32),
            pltpu.VMEM((ZR, F), _f32),
            pltpu.VMEM((ZR, 16), _f32),
        ] + [pltpu.SemaphoreType.DMA] * 11,
    )


def _att_mats(a_src, a_dst):
    eye = jnp.eye(H, dtype=_f32)
    asm = (a_src[:, :, None] * eye[:, None, :]).reshape(F, H)
    adm = (a_dst[:, :, None] * eye[:, None, :]).reshape(F, H)
    pad = jnp.zeros((F, 16 - H), _f32)
    return jnp.concatenate([asm, pad], axis=1), jnp.concatenate([adm, pad], axis=1)


def kernel(x, edge_index, W0, as0, ad0, b0, g0, be0, W1, as1, ad1, b1, g1, be1, Wc, bc):
    asm0, adm0 = _att_mats(as0, ad0)
    asm1, adm1 = _att_mats(as1, ad1)
    P = jnp.concatenate(
        [jnp.repeat(jnp.eye(H, dtype=_f32), HID, axis=1),
         jnp.zeros((16 - H, F), _f32)], axis=0)
    Q = jnp.tile(jnp.eye(HID, dtype=_f32) / H, (H, 1))
    Wcp = jnp.concatenate([Wc, jnp.zeros((HID, F - Wc.shape[1]), _f32)], axis=1)
    bcp = jnp.concatenate([bc, jnp.zeros((F - bc.shape[0],), _f32)])

    loops = jnp.arange(N, dtype=jnp.int32)
    padi = jnp.full((EPAD - EP,), N, jnp.int32)
    src2 = jnp.concatenate([edge_index[0].astype(jnp.int32), loops, padi])
    dst2 = jnp.concatenate([edge_index[1].astype(jnp.int32), loops, padi])

    sc_agg = _make_sc_agg()
    h0, aS0, aD0 = _tc_front(x, W0, asm0, adm0)
    n0, d0 = sc_agg(h0, aS0, aD0, src2, dst2)
    h1, aS1, aD1 = _tc_mid(n0, d0, b0, g0, be0, W1, asm1, adm1, P)
    n1, d1 = sc_agg(h1, aS1, aD1, src2, dst2)
    out = _tc_final(n1, d1, b1, g1, be1, P, Q, Wcp, bcp)
    return out[:, :2]
